# baseline (speedup 1.0000x reference)
import jax
import jax.numpy as jnp
from jax.experimental import pallas as pl
from jax.experimental.pallas import tpu as pltpu

NEG_INF = float("-inf")
BLK = 16


def _nms_body(x_ref, o_ref, rm_ref):
    x = x_ref[...]
    h = x.shape[1]
    left = jnp.concatenate([jnp.full_like(x[:, :, :1], NEG_INF), x[:, :, :-1]], axis=2)
    right = jnp.concatenate([x[:, :, 1:], jnp.full_like(x[:, :, :1], NEG_INF)], axis=2)
    rm = jnp.maximum(jnp.maximum(left, x), right)
    rm_ref[:, 8 : 8 + h, :] = rm
    rm_ref[:, 7:8, :] = rm[:, :1, :]
    rm_ref[:, 8 + h : 9 + h, :] = rm[:, h - 1 : h, :]
    hmax = jnp.maximum(
        jnp.maximum(rm_ref[:, 7 : 7 + h, :], rm_ref[:, 8 : 8 + h, :]),
        rm_ref[:, 9 : 9 + h, :],
    )
    o_ref[...] = jnp.where(hmax == x, x, 0.0)


def kernel(points):
    n, c, h, w = points.shape
    x = points.reshape(n * c, h, w)
    out = pl.pallas_call(
        _nms_body,
        grid=((n * c) // BLK,),
        in_specs=[pl.BlockSpec((BLK, h, w), lambda i: (i, 0, 0))],
        out_specs=pl.BlockSpec((BLK, h, w), lambda i: (i, 0, 0)),
        out_shape=jax.ShapeDtypeStruct((n * c, h, w), points.dtype),
        scratch_shapes=[pltpu.VMEM((BLK, h + 16, w), points.dtype)],
    )(x)
    return out.reshape(n, c, h, w)


# speedup vs baseline: 1.1729x; 1.1729x over previous
import jax
import jax.numpy as jnp
from jax.experimental import pallas as pl

NEG_INF = float("-inf")
BLK = 32
SUB = 8


def _nms_one(x):
    left = jnp.concatenate([jnp.full_like(x[:, :, :1], NEG_INF), x[:, :, :-1]], axis=2)
    right = jnp.concatenate([x[:, :, 1:], jnp.full_like(x[:, :, :1], NEG_INF)], axis=2)
    rowmax = jnp.maximum(jnp.maximum(left, x), right)
    up = jnp.concatenate([jnp.full_like(rowmax[:, :1, :], NEG_INF), rowmax[:, :-1, :]], axis=1)
    down = jnp.concatenate([rowmax[:, 1:, :], jnp.full_like(rowmax[:, :1, :], NEG_INF)], axis=1)
    hmax = jnp.maximum(jnp.maximum(up, rowmax), down)
    return jnp.where(hmax == x, x, 0.0)


def _nms_body(x_ref, o_ref):
    for s in range(BLK // SUB):
        x = x_ref[s * SUB : (s + 1) * SUB]
        o_ref[s * SUB : (s + 1) * SUB] = _nms_one(x)


def kernel(points):
    n, c, h, w = points.shape
    x = points.reshape(n * c, h, w)
    out = pl.pallas_call(
        _nms_body,
        grid=((n * c) // BLK,),
        in_specs=[pl.BlockSpec((BLK, h, w), lambda i: (i, 0, 0))],
        out_specs=pl.BlockSpec((BLK, h, w), lambda i: (i, 0, 0)),
        out_shape=jax.ShapeDtypeStruct((n * c, h, w), points.dtype),
    )(x)
    return out.reshape(n, c, h, w)


# speedup vs baseline: 1.2094x; 1.0311x over previous
import jax
import jax.numpy as jnp
from jax.experimental import pallas as pl

NEG_INF = float("-inf")
BLK = 32
SUB = 4


def _nms_one(x):
    left = jnp.concatenate([jnp.full_like(x[:, :, :1], NEG_INF), x[:, :, :-1]], axis=2)
    right = jnp.concatenate([x[:, :, 1:], jnp.full_like(x[:, :, :1], NEG_INF)], axis=2)
    rowmax = jnp.maximum(jnp.maximum(left, x), right)
    up = jnp.concatenate([jnp.full_like(rowmax[:, :1, :], NEG_INF), rowmax[:, :-1, :]], axis=1)
    down = jnp.concatenate([rowmax[:, 1:, :], jnp.full_like(rowmax[:, :1, :], NEG_INF)], axis=1)
    hmax = jnp.maximum(jnp.maximum(up, rowmax), down)
    return jnp.where(hmax == x, x, 0.0)


def _nms_body(x_ref, o_ref):
    for s in range(BLK // SUB):
        x = x_ref[s * SUB : (s + 1) * SUB]
        o_ref[s * SUB : (s + 1) * SUB] = _nms_one(x)


def kernel(points):
    n, c, h, w = points.shape
    x = points.reshape(n * c, h, w)
    out = pl.pallas_call(
        _nms_body,
        grid=((n * c) // BLK,),
        in_specs=[pl.BlockSpec((BLK, h, w), lambda i: (i, 0, 0))],
        out_specs=pl.BlockSpec((BLK, h, w), lambda i: (i, 0, 0)),
        out_shape=jax.ShapeDtypeStruct((n * c, h, w), points.dtype),
    )(x)
    return out.reshape(n, c, h, w)


# speedup vs baseline: 1.2610x; 1.0426x over previous
import jax
import jax.numpy as jnp
from jax.experimental import pallas as pl

NEG_INF = float("-inf")
BLK = 32
SUB = 2


def _nms_one(x):
    left = jnp.concatenate([jnp.full_like(x[:, :, :1], NEG_INF), x[:, :, :-1]], axis=2)
    right = jnp.concatenate([x[:, :, 1:], jnp.full_like(x[:, :, :1], NEG_INF)], axis=2)
    rowmax = jnp.maximum(jnp.maximum(left, x), right)
    up = jnp.concatenate([jnp.full_like(rowmax[:, :1, :], NEG_INF), rowmax[:, :-1, :]], axis=1)
    down = jnp.concatenate([rowmax[:, 1:, :], jnp.full_like(rowmax[:, :1, :], NEG_INF)], axis=1)
    hmax = jnp.maximum(jnp.maximum(up, rowmax), down)
    return jnp.where(hmax == x, x, 0.0)


def _nms_body(x_ref, o_ref):
    for s in range(BLK // SUB):
        x = x_ref[s * SUB : (s + 1) * SUB]
        o_ref[s * SUB : (s + 1) * SUB] = _nms_one(x)


def kernel(points):
    n, c, h, w = points.shape
    x = points.reshape(n * c, h, w)
    out = pl.pallas_call(
        _nms_body,
        grid=((n * c) // BLK,),
        in_specs=[pl.BlockSpec((BLK, h, w), lambda i: (i, 0, 0))],
        out_specs=pl.BlockSpec((BLK, h, w), lambda i: (i, 0, 0)),
        out_shape=jax.ShapeDtypeStruct((n * c, h, w), points.dtype),
    )(x)
    return out.reshape(n, c, h, w)


# speedup vs baseline: 1.2664x; 1.0043x over previous
import jax
import jax.numpy as jnp
from jax.experimental import pallas as pl

NEG_INF = float("-inf")
BLK = 32
SUB = 1


def _nms_one(x):
    left = jnp.concatenate([jnp.full_like(x[:, :, :1], NEG_INF), x[:, :, :-1]], axis=2)
    right = jnp.concatenate([x[:, :, 1:], jnp.full_like(x[:, :, :1], NEG_INF)], axis=2)
    rowmax = jnp.maximum(jnp.maximum(left, x), right)
    up = jnp.concatenate([jnp.full_like(rowmax[:, :1, :], NEG_INF), rowmax[:, :-1, :]], axis=1)
    down = jnp.concatenate([rowmax[:, 1:, :], jnp.full_like(rowmax[:, :1, :], NEG_INF)], axis=1)
    hmax = jnp.maximum(jnp.maximum(up, rowmax), down)
    return jnp.where(hmax == x, x, 0.0)


def _nms_body(x_ref, o_ref):
    for s in range(BLK // SUB):
        x = x_ref[s * SUB : (s + 1) * SUB]
        o_ref[s * SUB : (s + 1) * SUB] = _nms_one(x)


def kernel(points):
    n, c, h, w = points.shape
    x = points.reshape(n * c, h, w)
    out = pl.pallas_call(
        _nms_body,
        grid=((n * c) // BLK,),
        in_specs=[pl.BlockSpec((BLK, h, w), lambda i: (i, 0, 0))],
        out_specs=pl.BlockSpec((BLK, h, w), lambda i: (i, 0, 0)),
        out_shape=jax.ShapeDtypeStruct((n * c, h, w), points.dtype),
    )(x)
    return out.reshape(n, c, h, w)
